# baseline (speedup 1.0000x reference)
import functools

import jax
import jax.numpy as jnp
from jax import lax
from jax.experimental import pallas as pl
from jax.experimental.pallas import tpu as pltpu
from jax.experimental.pallas import tpu_sc as plsc

NC, NS, L = 2, 16, 16
NW = NC * NS
B = 16384
D = 128
BPW = B // NW
C = 64
G = BPW // C
NB = 2

SECONDS_PER_DAY = 86400.0
P_COEF = (6.944574454e-05, 0.9962619482, -0.4664424386, 0.2186654837,
          -0.05545931374)

_BCAST_DNUMS = lax.GatherDimensionNumbers(
    offset_dims=(), collapsed_slice_dims=(0,), start_index_map=(0,))


def _lane_bcast(v16, k):
    idx = jnp.full((L, 1), k, jnp.int32)
    return lax.gather(v16, idx, _BCAST_DNUMS, (1,),
                      mode=lax.GatherScatterMode.PROMISE_IN_BOUNDS)


def _decay_body(ids_hbm, dt_hbm, rc_hbm, prof_hbm, table_hbm, av_hbm, gv_hbm,
                out_hbm,
                idx_all, rows_v, dt_v, rc_v, prof_v, out_v, av_v, gv_v,
                in_sem0, in_sem1, out_sem0, out_sem1):
    wid = lax.axis_index("s") * NC + lax.axis_index("c")
    base = wid * BPW
    in_sems = (in_sem0, in_sem1)
    out_sems = (out_sem0, out_sem1)

    pltpu.sync_copy(ids_hbm.at[pl.ds(base, BPW)], idx_all)
    pltpu.sync_copy(av_hbm, av_v)
    pltpu.sync_copy(gv_hbm, gv_v)
    av = av_v[...]
    gv = gv_v[...]

    def input_copies(g, nb):
        r0 = base + g * C
        s = in_sems[nb]
        return [
            pltpu.make_async_copy(
                table_hbm.at[idx_all.at[pl.ds(g * C, C)]], rows_v.at[nb], s),
            pltpu.make_async_copy(dt_hbm.at[pl.ds(r0, C)], dt_v.at[nb], s),
            pltpu.make_async_copy(rc_hbm.at[pl.ds(r0, C)], rc_v.at[nb], s),
            pltpu.make_async_copy(prof_hbm.at[pl.ds(r0, C)], prof_v.at[nb], s),
        ]

    def out_copy(g, nb):
        return pltpu.make_async_copy(
            out_v.at[nb], out_hbm.at[pl.ds(base + g * C, C)], out_sems[nb])

    def compute(nb):
        @plsc.parallel_loop(0, C, unroll=4)
        def row_body(r):
            pv = prof_v[nb, r, :]
            prow = 1.0 + gv * jnp.clip(pv, 0.0, 1.0)
            for j in range(D // L):
                sl = pl.ds(j * L, L)
                lam = jnp.clip(rows_v[nb, r, sl], 0.005, 0.05)
                t = rc_v[nb, r, sl]
                p = jnp.float32(P_COEF[4])
                for c in (3, 2, 1, 0):
                    p = p * t + P_COEF[c]
                denom = (1.0 + av * p) * prow
                z = lam * dt_v[nb, r, sl] * (-1.0 / SECONDS_PER_DAY)
                out_v[nb, r, sl] = jnp.exp(z / denom)

    for c in input_copies(0, 0):
        c.start()
    for c in input_copies(1, 1):
        c.start()

    def chunk_pair(k, carry):
        for h in range(NB):
            g = NB * k + h
            for c in input_copies(g, h):
                c.wait()

            @pl.when(g >= NB)
            def _():
                out_copy(g - NB, h).wait()

            compute(h)
            out_copy(g, h).start()

            @pl.when(g + NB < G)
            def _():
                for c in input_copies(g + NB, h):
                    c.start()
        return carry

    lax.fori_loop(0, G // NB, chunk_pair, 0)
    for g in range(G - NB, G):
        out_copy(g, g % NB).wait()


_decay_call = pl.kernel(
    _decay_body,
    out_type=jax.ShapeDtypeStruct((B, D), jnp.float32),
    mesh=plsc.VectorSubcoreMesh(core_axis_name="c", subcore_axis_name="s"),
    scratch_types=[
        pltpu.VMEM((BPW,), jnp.int32),
        pltpu.VMEM((NB, C, D), jnp.float32),
        pltpu.VMEM((NB, C, D), jnp.float32),
        pltpu.VMEM((NB, C, D), jnp.float32),
        pltpu.VMEM((NB, C, L), jnp.float32),
        pltpu.VMEM((NB, C, D), jnp.float32),
        pltpu.VMEM((L,), jnp.float32),
        pltpu.VMEM((L,), jnp.float32),
        pltpu.SemaphoreType.DMA,
        pltpu.SemaphoreType.DMA,
        pltpu.SemaphoreType.DMA,
        pltpu.SemaphoreType.DMA,
    ],
)


def kernel(student_ids, delta_t, review_count, proficiency, lambda_table,
           alpha_logit, gamma_logit):
    alpha = jax.nn.sigmoid(alpha_logit) * 1.9 + 0.1
    gamma = jax.nn.sigmoid(gamma_logit) * 2.9 + 0.1
    av = jnp.full((L,), alpha, jnp.float32)
    gv = jnp.full((L,), gamma, jnp.float32)
    ids = student_ids.astype(jnp.int32)
    prof_b = jnp.broadcast_to(proficiency[:, None], (B, L))
    return _decay_call(ids, delta_t, review_count, prof_b,
                       lambda_table, av, gv)


# speedup vs baseline: 1.2502x; 1.2502x over previous
import functools

import jax
import jax.numpy as jnp
from jax import lax
from jax.experimental import pallas as pl
from jax.experimental.pallas import tpu as pltpu
from jax.experimental.pallas import tpu_sc as plsc

NC, NS, L = 2, 16, 16
NW = NC * NS
B = 16384
D = 128
BPW = B // NW
C = 64
G = BPW // C
NB = 2

SECONDS_PER_DAY = 86400.0
P_COEF = (6.944574454e-05, 0.9962619482, -0.4664424386, 0.2186654837,
          -0.05545931374)

_BCAST_DNUMS = lax.GatherDimensionNumbers(
    offset_dims=(), collapsed_slice_dims=(0,), start_index_map=(0,))


def _lane_bcast(v16, k):
    idx = jnp.full((L, 1), k, jnp.int32)
    return lax.gather(v16, idx, _BCAST_DNUMS, (1,),
                      mode=lax.GatherScatterMode.PROMISE_IN_BOUNDS)


def _decay_body(ids_hbm, dt_hbm, rc_hbm, prof_hbm, table_hbm, av_hbm, gv_hbm,
                out_hbm,
                idx_all, rows_v, dt_v, rc_v, prof_v, out_v, av_v, gv_v,
                in_sem0, in_sem1, out_sem0, out_sem1):
    wid = lax.axis_index("s") * NC + lax.axis_index("c")
    base = wid * BPW
    in_sems = (in_sem0, in_sem1)
    out_sems = (out_sem0, out_sem1)

    pltpu.sync_copy(ids_hbm.at[pl.ds(base, BPW)], idx_all)
    pltpu.sync_copy(av_hbm, av_v)
    pltpu.sync_copy(gv_hbm, gv_v)
    av = av_v[...]
    gv = gv_v[...]

    def input_copies(g, nb):
        r0 = base + g * C
        s = in_sems[nb]
        return [
            pltpu.make_async_copy(
                table_hbm.at[idx_all.at[pl.ds(g * C, C)]], rows_v.at[nb], s),
            pltpu.make_async_copy(dt_hbm.at[pl.ds(r0, C)], dt_v.at[nb], s),
            pltpu.make_async_copy(rc_hbm.at[pl.ds(r0, C)], rc_v.at[nb], s),
            pltpu.make_async_copy(prof_hbm.at[pl.ds(r0, C)], prof_v.at[nb], s),
        ]

    def out_copy(g, nb):
        return pltpu.make_async_copy(
            out_v.at[nb], out_hbm.at[pl.ds(base + g * C, C)], out_sems[nb])

    def compute(nb):
        @plsc.parallel_loop(0, C, unroll=1)
        def row_body(r):
            pv = prof_v[nb, r, :]
            prow = 1.0 + gv * jnp.clip(pv, 0.0, 1.0)
            for j in range(D // L):
                sl = pl.ds(j * L, L)
                lam = jnp.clip(rows_v[nb, r, sl], 0.005, 0.05)
                t = rc_v[nb, r, sl]
                p = jnp.float32(P_COEF[4])
                for c in (3, 2, 1, 0):
                    p = p * t + P_COEF[c]
                denom = (1.0 + av * p) * prow
                z = lam * dt_v[nb, r, sl] * (-1.0 / SECONDS_PER_DAY)
                out_v[nb, r, sl] = jnp.exp(z / denom)

    for c in input_copies(0, 0):
        c.start()
    for c in input_copies(1, 1):
        c.start()

    def chunk_pair(k, carry):
        for h in range(NB):
            g = NB * k + h
            for c in input_copies(g, h):
                c.wait()

            @pl.when(g >= NB)
            def _():
                out_copy(g - NB, h).wait()

            compute(h)
            out_copy(g, h).start()

            @pl.when(g + NB < G)
            def _():
                for c in input_copies(g + NB, h):
                    c.start()
        return carry

    lax.fori_loop(0, G // NB, chunk_pair, 0)
    for g in range(G - NB, G):
        out_copy(g, g % NB).wait()


_decay_call = pl.kernel(
    _decay_body,
    out_type=jax.ShapeDtypeStruct((B, D), jnp.float32),
    mesh=plsc.VectorSubcoreMesh(core_axis_name="c", subcore_axis_name="s"),
    scratch_types=[
        pltpu.VMEM((BPW,), jnp.int32),
        pltpu.VMEM((NB, C, D), jnp.float32),
        pltpu.VMEM((NB, C, D), jnp.float32),
        pltpu.VMEM((NB, C, D), jnp.float32),
        pltpu.VMEM((NB, C, L), jnp.float32),
        pltpu.VMEM((NB, C, D), jnp.float32),
        pltpu.VMEM((L,), jnp.float32),
        pltpu.VMEM((L,), jnp.float32),
        pltpu.SemaphoreType.DMA,
        pltpu.SemaphoreType.DMA,
        pltpu.SemaphoreType.DMA,
        pltpu.SemaphoreType.DMA,
    ],
)


def kernel(student_ids, delta_t, review_count, proficiency, lambda_table,
           alpha_logit, gamma_logit):
    alpha = jax.nn.sigmoid(alpha_logit) * 1.9 + 0.1
    gamma = jax.nn.sigmoid(gamma_logit) * 2.9 + 0.1
    av = jnp.full((L,), alpha, jnp.float32)
    gv = jnp.full((L,), gamma, jnp.float32)
    ids = student_ids.astype(jnp.int32)
    prof_b = jnp.broadcast_to(proficiency[:, None], (B, L))
    return _decay_call(ids, delta_t, review_count, prof_b,
                       lambda_table, av, gv)


# speedup vs baseline: 1.6392x; 1.3111x over previous
import functools

import jax
import jax.numpy as jnp
from jax import lax
from jax.experimental import pallas as pl
from jax.experimental.pallas import tpu as pltpu
from jax.experimental.pallas import tpu_sc as plsc

NC, NS, L = 2, 16, 16
NW = NC * NS
B = 16384
D = 128
BPW = B // NW
C = 128
G = BPW // C
NB = 2

R = 8192
SECONDS_PER_DAY = 86400.0


def _gather_body(ids_hbm, table_hbm, out_hbm,
                 idx_v, rows_v, isem, gsem0, gsem1, osem0, osem1):
    wid = lax.axis_index("s") * NC + lax.axis_index("c")
    base = wid * BPW
    gsems = (gsem0, gsem1)
    osems = (osem0, osem1)
    ih = [None] * G
    gh = [None] * G
    oh = [None] * G

    def start_idx(g):
        ih[g] = pltpu.async_copy(
            ids_hbm.at[pl.ds(base + g * C, C)], idx_v.at[g % NB], isem)

    def start_gather(g):
        nb = g % NB
        gh[g] = pltpu.async_copy(
            table_hbm.at[idx_v.at[nb]], rows_v.at[nb], gsems[nb])

    start_idx(0)
    if G > 1:
        start_idx(1)
    ih[0].wait()
    start_gather(0)
    for g in range(G):
        nb = g % NB
        if g + 1 < G:
            ih[g + 1].wait()
            start_gather(g + 1)
        if g + NB < G:
            start_idx(g + NB)
        gh[g].wait()
        if g >= NB:
            oh[g - NB].wait()
        oh[g] = pltpu.async_copy(
            rows_v.at[nb], out_hbm.at[pl.ds(base + g * C, C)], osems[nb])
    for g in range(max(0, G - NB), G):
        oh[g].wait()


_gather_call = pl.kernel(
    _gather_body,
    out_type=jax.ShapeDtypeStruct((B, D), jnp.float32),
    mesh=plsc.VectorSubcoreMesh(core_axis_name="c", subcore_axis_name="s"),
    scratch_types=[
        pltpu.VMEM((NB, C), jnp.int32),
        pltpu.VMEM((NB, C, D), jnp.float32),
        pltpu.SemaphoreType.DMA,
        pltpu.SemaphoreType.DMA,
        pltpu.SemaphoreType.DMA,
        pltpu.SemaphoreType.DMA,
        pltpu.SemaphoreType.DMA,
    ],
)


def _math_body(ab_ref, lam_ref, dt_ref, rc_ref, prof_ref, out_ref):
    a = ab_ref[0, 0]
    g = ab_ref[0, 1]
    lam = jnp.clip(lam_ref[...], 0.005, 0.05)
    denom = (1.0 + a * jnp.log1p(rc_ref[...])) \
        * (1.0 + g * jnp.clip(prof_ref[...], 0.0, 1.0))[:, None]
    z = lam * dt_ref[...] * (-1.0 / SECONDS_PER_DAY)
    out_ref[...] = jnp.exp(z / denom)


_math_call = pl.pallas_call(
    _math_body,
    out_shape=jax.ShapeDtypeStruct((B, D), jnp.float32),
    grid=(B // R,),
    in_specs=[
        pl.BlockSpec(memory_space=pltpu.SMEM),
        pl.BlockSpec((R, D), lambda i: (i, 0)),
        pl.BlockSpec((R, D), lambda i: (i, 0)),
        pl.BlockSpec((R, D), lambda i: (i, 0)),
        pl.BlockSpec((R,), lambda i: (i,)),
    ],
    out_specs=pl.BlockSpec((R, D), lambda i: (i, 0)),
)


def kernel(student_ids, delta_t, review_count, proficiency, lambda_table,
           alpha_logit, gamma_logit):
    alpha = jax.nn.sigmoid(alpha_logit) * 1.9 + 0.1
    gamma = jax.nn.sigmoid(gamma_logit) * 2.9 + 0.1
    ab = jnp.stack([alpha, gamma]).reshape(1, 2)
    ids = student_ids.astype(jnp.int32)
    lam = _gather_call(ids, lambda_table)
    return _math_call(ab, lam, delta_t, review_count, proficiency)
